# baseline (device time: 154237 ns/iter reference)
import jax
import jax.numpy as jnp
from jax import lax
from jax.experimental import pallas as pl
from jax.experimental.pallas import tpu as pltpu

N_Z = 4


def kernel(partial, gamma):
    _, m_total, d = partial.shape
    m_per = m_total // N_Z
    x = jnp.reshape(partial, (m_total, d))
    g = jnp.reshape(gamma, (1, d))

    def body(x_ref, g_ref, out_ref, comm_ref, send_sems, recv_sems):
        my_x = lax.axis_index("x")
        my_y = lax.axis_index("y")
        my_z = lax.axis_index("z")
        up = lax.rem(my_z + 1, N_Z)
        down = lax.rem(my_z + N_Z - 1, N_Z)

        barrier_sem = pltpu.get_barrier_semaphore()
        for nbr in (up, down):
            pl.semaphore_signal(
                barrier_sem,
                inc=1,
                device_id=(my_x, my_y, nbr),
                device_id_type=pl.DeviceIdType.MESH,
            )
        pl.semaphore_wait(barrier_sem, 2)

        c = lax.rem(my_z + N_Z - 1, N_Z)
        comm_ref[0] = x_ref[pl.ds(c * m_per, m_per), :]

        for s in range(N_Z - 1):
            send_slot = s % 2
            recv_slot = (s + 1) % 2
            rdma = pltpu.make_async_remote_copy(
                src_ref=comm_ref.at[send_slot],
                dst_ref=comm_ref.at[recv_slot],
                send_sem=send_sems.at[send_slot],
                recv_sem=recv_sems.at[recv_slot],
                device_id=(my_x, my_y, up),
                device_id_type=pl.DeviceIdType.MESH,
            )
            rdma.start()
            rdma.wait()
            c = lax.rem(my_z + 2 * N_Z - 2 - s, N_Z)
            acc = comm_ref[recv_slot] + x_ref[pl.ds(c * m_per, m_per), :]
            if s < N_Z - 2:
                comm_ref[recv_slot] = acc
            else:
                rms = jnp.sqrt(
                    jnp.mean(acc * acc, axis=-1, keepdims=True) + 1e-6
                )
                out_ref[...] = acc / rms * g_ref[...]

    return pl.pallas_call(
        body,
        out_shape=jax.ShapeDtypeStruct((m_per, d), jnp.float32),
        in_specs=[
            pl.BlockSpec(memory_space=pltpu.VMEM),
            pl.BlockSpec(memory_space=pltpu.VMEM),
        ],
        out_specs=pl.BlockSpec(memory_space=pltpu.VMEM),
        scratch_shapes=[
            pltpu.VMEM((2, m_per, d), jnp.float32),
            pltpu.SemaphoreType.DMA((2,)),
            pltpu.SemaphoreType.DMA((2,)),
        ],
        compiler_params=pltpu.CompilerParams(collective_id=0),
    )(x, g)


# device time: 153105 ns/iter; 1.0074x vs baseline; 1.0074x over previous
import jax
import jax.numpy as jnp
from jax import lax
from jax.experimental import pallas as pl
from jax.experimental.pallas import tpu as pltpu

N_Z = 4


def kernel(partial, gamma):
    _, m_total, d = partial.shape
    m_per = m_total // N_Z
    m_half = m_per // 2
    x = jnp.reshape(partial, (m_total, d))
    g = jnp.reshape(gamma, (1, d))

    def body(
        x_ref,
        g_ref,
        out_ref,
        up_ref,
        dn_ref,
        up_send_sems,
        up_recv_sems,
        dn_send_sems,
        dn_recv_sems,
    ):
        my_x = lax.axis_index("x")
        my_y = lax.axis_index("y")
        my_z = lax.axis_index("z")
        up = lax.rem(my_z + 1, N_Z)
        down = lax.rem(my_z + N_Z - 1, N_Z)

        barrier_sem = pltpu.get_barrier_semaphore()
        for nbr in (up, down):
            pl.semaphore_signal(
                barrier_sem,
                inc=1,
                device_id=(my_x, my_y, nbr),
                device_id_type=pl.DeviceIdType.MESH,
            )
        pl.semaphore_wait(barrier_sem, 2)

        c_up = lax.rem(my_z + N_Z - 1, N_Z)
        c_dn = lax.rem(my_z + 1, N_Z)
        up_ref[0] = x_ref[pl.ds(c_up * m_per, m_half), :]
        dn_ref[0] = x_ref[pl.ds(c_dn * m_per + m_half, m_half), :]

        for s in range(N_Z - 1):
            send_slot = s % 2
            recv_slot = (s + 1) % 2
            rdma_up = pltpu.make_async_remote_copy(
                src_ref=up_ref.at[send_slot],
                dst_ref=up_ref.at[recv_slot],
                send_sem=up_send_sems.at[send_slot],
                recv_sem=up_recv_sems.at[recv_slot],
                device_id=(my_x, my_y, up),
                device_id_type=pl.DeviceIdType.MESH,
            )
            rdma_dn = pltpu.make_async_remote_copy(
                src_ref=dn_ref.at[send_slot],
                dst_ref=dn_ref.at[recv_slot],
                send_sem=dn_send_sems.at[send_slot],
                recv_sem=dn_recv_sems.at[recv_slot],
                device_id=(my_x, my_y, down),
                device_id_type=pl.DeviceIdType.MESH,
            )
            rdma_up.start()
            rdma_dn.start()
            rdma_up.wait()
            rdma_dn.wait()
            c_up = lax.rem(my_z + 2 * N_Z - 2 - s, N_Z)
            c_dn = lax.rem(my_z + 2 + s, N_Z)
            acc_up = up_ref[recv_slot] + x_ref[pl.ds(c_up * m_per, m_half), :]
            acc_dn = dn_ref[recv_slot] + x_ref[
                pl.ds(c_dn * m_per + m_half, m_half), :
            ]
            if s < N_Z - 2:
                up_ref[recv_slot] = acc_up
                dn_ref[recv_slot] = acc_dn
            else:
                rms_up = jnp.sqrt(
                    jnp.mean(acc_up * acc_up, axis=-1, keepdims=True) + 1e-6
                )
                rms_dn = jnp.sqrt(
                    jnp.mean(acc_dn * acc_dn, axis=-1, keepdims=True) + 1e-6
                )
                out_ref[pl.ds(0, m_half), :] = acc_up / rms_up * g_ref[...]
                out_ref[pl.ds(m_half, m_half), :] = acc_dn / rms_dn * g_ref[...]

    return pl.pallas_call(
        body,
        out_shape=jax.ShapeDtypeStruct((m_per, d), jnp.float32),
        in_specs=[
            pl.BlockSpec(memory_space=pltpu.VMEM),
            pl.BlockSpec(memory_space=pltpu.VMEM),
        ],
        out_specs=pl.BlockSpec(memory_space=pltpu.VMEM),
        scratch_shapes=[
            pltpu.VMEM((2, m_half, d), jnp.float32),
            pltpu.VMEM((2, m_half, d), jnp.float32),
            pltpu.SemaphoreType.DMA((2,)),
            pltpu.SemaphoreType.DMA((2,)),
            pltpu.SemaphoreType.DMA((2,)),
            pltpu.SemaphoreType.DMA((2,)),
        ],
        compiler_params=pltpu.CompilerParams(collective_id=0),
    )(x, g)


# device time: 78994 ns/iter; 1.9525x vs baseline; 1.9382x over previous
import jax
import jax.numpy as jnp
from jax import lax
from jax.experimental import pallas as pl
from jax.experimental.pallas import tpu as pltpu

N_Z = 4
N_Q = 4


def kernel(partial, gamma):
    _, m_total, d = partial.shape
    m_per = m_total // N_Z
    m_half = m_per // 2
    dq = d // N_Q
    x = jnp.reshape(partial, (m_total, d))
    g = jnp.reshape(gamma, (1, d))

    def body(
        x_ref,
        g_ref,
        out_ref,
        xq_ref,
        up_ref,
        dn_ref,
        agg_ref,
        gx_ref,
        up_send_sems,
        up_recv_sems,
        dn_send_sems,
        dn_recv_sems,
        ag_send_sems,
        ag_recv_sems,
    ):
        my_x = lax.axis_index("x")
        my_y = lax.axis_index("y")
        my_z = lax.axis_index("z")
        up = lax.rem(my_z + 1, N_Z)
        down = lax.rem(my_z + N_Z - 1, N_Z)
        q = my_x * 2 + my_y

        partners = (
            (my_x, my_y, up),
            (my_x, my_y, down),
            (1 - my_x, my_y, my_z),
            (my_x, 1 - my_y, my_z),
        )
        barrier_sem = pltpu.get_barrier_semaphore()
        for dev in partners:
            pl.semaphore_signal(
                barrier_sem,
                inc=1,
                device_id=dev,
                device_id_type=pl.DeviceIdType.MESH,
            )
        pl.semaphore_wait(barrier_sem, len(partners))

        for qq in range(N_Q):
            @pl.when(q == qq)
            def _():
                xq_ref[...] = x_ref[:, qq * dq:(qq + 1) * dq]

        c_up = lax.rem(my_z + N_Z - 1, N_Z)
        c_dn = lax.rem(my_z + 1, N_Z)
        up_ref[0] = xq_ref[pl.ds(c_up * m_per, m_half), :]
        dn_ref[0] = xq_ref[pl.ds(c_dn * m_per + m_half, m_half), :]

        for s in range(N_Z - 1):
            send_slot = s % 2
            recv_slot = (s + 1) % 2
            rdma_up = pltpu.make_async_remote_copy(
                src_ref=up_ref.at[send_slot],
                dst_ref=up_ref.at[recv_slot],
                send_sem=up_send_sems.at[send_slot],
                recv_sem=up_recv_sems.at[recv_slot],
                device_id=(my_x, my_y, up),
                device_id_type=pl.DeviceIdType.MESH,
            )
            rdma_dn = pltpu.make_async_remote_copy(
                src_ref=dn_ref.at[send_slot],
                dst_ref=dn_ref.at[recv_slot],
                send_sem=dn_send_sems.at[send_slot],
                recv_sem=dn_recv_sems.at[recv_slot],
                device_id=(my_x, my_y, down),
                device_id_type=pl.DeviceIdType.MESH,
            )
            rdma_up.start()
            rdma_dn.start()
            rdma_up.wait()
            rdma_dn.wait()
            c_up = lax.rem(my_z + 2 * N_Z - 2 - s, N_Z)
            c_dn = lax.rem(my_z + 2 + s, N_Z)
            acc_up = up_ref[recv_slot] + xq_ref[pl.ds(c_up * m_per, m_half), :]
            acc_dn = dn_ref[recv_slot] + xq_ref[
                pl.ds(c_dn * m_per + m_half, m_half), :
            ]
            if s < N_Z - 2:
                up_ref[recv_slot] = acc_up
                dn_ref[recv_slot] = acc_dn
            else:
                agg_ref[pl.ds(0, m_half), :] = acc_up
                agg_ref[pl.ds(m_half, m_half), :] = acc_dn

        ag_x = pltpu.make_async_remote_copy(
            src_ref=agg_ref,
            dst_ref=gx_ref.at[0],
            send_sem=ag_send_sems.at[0],
            recv_sem=ag_recv_sems.at[0],
            device_id=(1 - my_x, my_y, my_z),
            device_id_type=pl.DeviceIdType.MESH,
        )
        ag_y = pltpu.make_async_remote_copy(
            src_ref=agg_ref,
            dst_ref=gx_ref.at[1],
            send_sem=ag_send_sems.at[1],
            recv_sem=ag_recv_sems.at[1],
            device_id=(my_x, 1 - my_y, my_z),
            device_id_type=pl.DeviceIdType.MESH,
        )
        ag_x.start()
        ag_y.start()
        ag_x.wait()
        ag_y.wait()
        ag_d = pltpu.make_async_remote_copy(
            src_ref=gx_ref.at[0],
            dst_ref=gx_ref.at[2],
            send_sem=ag_send_sems.at[2],
            recv_sem=ag_recv_sems.at[2],
            device_id=(my_x, 1 - my_y, my_z),
            device_id_type=pl.DeviceIdType.MESH,
        )
        ag_d.start()
        ag_d.wait()

        mine = agg_ref[...]
        from_x = gx_ref[0]
        from_y = gx_ref[1]
        diag = gx_ref[2]
        ssq = (
            jnp.sum(mine * mine, axis=-1, keepdims=True)
            + jnp.sum(from_x * from_x, axis=-1, keepdims=True)
            + jnp.sum(from_y * from_y, axis=-1, keepdims=True)
            + jnp.sum(diag * diag, axis=-1, keepdims=True)
        )
        inv_rms = lax.rsqrt(ssq / d + 1e-6)

        for qq in range(N_Q):
            @pl.when(q == qq)
            def _():
                for quarter, dst in (
                    (mine, qq),
                    (from_x, qq ^ 2),
                    (from_y, qq ^ 1),
                    (diag, qq ^ 3),
                ):
                    sl = slice(dst * dq, (dst + 1) * dq)
                    out_ref[:, sl] = quarter * inv_rms * g_ref[:, sl]

    return pl.pallas_call(
        body,
        out_shape=jax.ShapeDtypeStruct((m_per, d), jnp.float32),
        in_specs=[
            pl.BlockSpec(memory_space=pltpu.VMEM),
            pl.BlockSpec(memory_space=pltpu.VMEM),
        ],
        out_specs=pl.BlockSpec(memory_space=pltpu.VMEM),
        scratch_shapes=[
            pltpu.VMEM((m_total, dq), jnp.float32),
            pltpu.VMEM((2, m_half, dq), jnp.float32),
            pltpu.VMEM((2, m_half, dq), jnp.float32),
            pltpu.VMEM((m_per, dq), jnp.float32),
            pltpu.VMEM((3, m_per, dq), jnp.float32),
            pltpu.SemaphoreType.DMA((2,)),
            pltpu.SemaphoreType.DMA((2,)),
            pltpu.SemaphoreType.DMA((2,)),
            pltpu.SemaphoreType.DMA((2,)),
            pltpu.SemaphoreType.DMA((3,)),
            pltpu.SemaphoreType.DMA((3,)),
        ],
        compiler_params=pltpu.CompilerParams(collective_id=0),
    )(x, g)


# device time: 74033 ns/iter; 2.0834x vs baseline; 1.0670x over previous
import jax
import jax.numpy as jnp
from jax import lax
from jax.experimental import pallas as pl
from jax.experimental.pallas import tpu as pltpu

N_Z = 4
N_Q = 4
N_R = 2


def kernel(partial, gamma):
    _, m_total, d = partial.shape
    m_per = m_total // N_Z
    mr = m_per // N_R
    mh = mr // 2
    dq = d // N_Q
    x = jnp.reshape(partial, (m_total, d))
    g = jnp.reshape(gamma, (1, d))

    def body(
        x_ref,
        g_ref,
        out_ref,
        xq_ref,
        up_ref,
        dn_ref,
        agg_ref,
        gx_ref,
        up_send_sems,
        up_recv_sems,
        dn_send_sems,
        dn_recv_sems,
        ag_send_sems,
        ag_recv_sems,
    ):
        my_x = lax.axis_index("x")
        my_y = lax.axis_index("y")
        my_z = lax.axis_index("z")
        up = lax.rem(my_z + 1, N_Z)
        down = lax.rem(my_z + N_Z - 1, N_Z)
        q = my_x * 2 + my_y
        x_nbr = (1 - my_x, my_y, my_z)
        y_nbr = (my_x, 1 - my_y, my_z)

        partners = ((my_x, my_y, up), (my_x, my_y, down), x_nbr, y_nbr)
        barrier_sem = pltpu.get_barrier_semaphore()
        for dev in partners:
            pl.semaphore_signal(
                barrier_sem,
                inc=1,
                device_id=dev,
                device_id_type=pl.DeviceIdType.MESH,
            )
        pl.semaphore_wait(barrier_sem, len(partners))

        for qq in range(N_Q):
            @pl.when(q == qq)
            def _():
                xq_ref[...] = x_ref[:, qq * dq:(qq + 1) * dq]

        def ring_round(r):
            c_up = lax.rem(my_z + N_Z - 1, N_Z)
            c_dn = lax.rem(my_z + 1, N_Z)
            up_ref[r, 0] = xq_ref[pl.ds(c_up * m_per + r * mr, mh), :]
            dn_ref[r, 0] = xq_ref[pl.ds(c_dn * m_per + r * mr + mh, mh), :]
            for s in range(N_Z - 1):
                send_slot = s % 2
                recv_slot = (s + 1) % 2
                rdma_up = pltpu.make_async_remote_copy(
                    src_ref=up_ref.at[r, send_slot],
                    dst_ref=up_ref.at[r, recv_slot],
                    send_sem=up_send_sems.at[r, send_slot],
                    recv_sem=up_recv_sems.at[r, recv_slot],
                    device_id=(my_x, my_y, up),
                    device_id_type=pl.DeviceIdType.MESH,
                )
                rdma_dn = pltpu.make_async_remote_copy(
                    src_ref=dn_ref.at[r, send_slot],
                    dst_ref=dn_ref.at[r, recv_slot],
                    send_sem=dn_send_sems.at[r, send_slot],
                    recv_sem=dn_recv_sems.at[r, recv_slot],
                    device_id=(my_x, my_y, down),
                    device_id_type=pl.DeviceIdType.MESH,
                )
                rdma_up.start()
                rdma_dn.start()
                rdma_up.wait()
                rdma_dn.wait()
                c_up = lax.rem(my_z + 2 * N_Z - 2 - s, N_Z)
                c_dn = lax.rem(my_z + 2 + s, N_Z)
                acc_up = up_ref[r, recv_slot] + xq_ref[
                    pl.ds(c_up * m_per + r * mr, mh), :
                ]
                acc_dn = dn_ref[r, recv_slot] + xq_ref[
                    pl.ds(c_dn * m_per + r * mr + mh, mh), :
                ]
                if s < N_Z - 2:
                    up_ref[r, recv_slot] = acc_up
                    dn_ref[r, recv_slot] = acc_dn
                else:
                    agg_ref[r, pl.ds(0, mh), :] = acc_up
                    agg_ref[r, pl.ds(mh, mh), :] = acc_dn

        def ag_swap(r):
            ag_x = pltpu.make_async_remote_copy(
                src_ref=agg_ref.at[r],
                dst_ref=gx_ref.at[r, 0],
                send_sem=ag_send_sems.at[r, 0],
                recv_sem=ag_recv_sems.at[r, 0],
                device_id=x_nbr,
                device_id_type=pl.DeviceIdType.MESH,
            )
            ag_y = pltpu.make_async_remote_copy(
                src_ref=agg_ref.at[r],
                dst_ref=gx_ref.at[r, 1],
                send_sem=ag_send_sems.at[r, 1],
                recv_sem=ag_recv_sems.at[r, 1],
                device_id=y_nbr,
                device_id_type=pl.DeviceIdType.MESH,
            )
            return ag_x, ag_y

        def ag_diag(r):
            d_y = pltpu.make_async_remote_copy(
                src_ref=gx_ref.at[r, 0, pl.ds(0, mh)],
                dst_ref=gx_ref.at[r, 2, pl.ds(0, mh)],
                send_sem=ag_send_sems.at[r, 2],
                recv_sem=ag_recv_sems.at[r, 2],
                device_id=y_nbr,
                device_id_type=pl.DeviceIdType.MESH,
            )
            d_x = pltpu.make_async_remote_copy(
                src_ref=gx_ref.at[r, 1, pl.ds(mh, mh)],
                dst_ref=gx_ref.at[r, 2, pl.ds(mh, mh)],
                send_sem=ag_send_sems.at[r, 3],
                recv_sem=ag_recv_sems.at[r, 3],
                device_id=x_nbr,
                device_id_type=pl.DeviceIdType.MESH,
            )
            return d_y, d_x

        def assemble(r):
            mine = agg_ref[r]
            from_x = gx_ref[r, 0]
            from_y = gx_ref[r, 1]
            diag = gx_ref[r, 2]
            ssq = (
                jnp.sum(mine * mine, axis=-1, keepdims=True)
                + jnp.sum(from_x * from_x, axis=-1, keepdims=True)
                + jnp.sum(from_y * from_y, axis=-1, keepdims=True)
                + jnp.sum(diag * diag, axis=-1, keepdims=True)
            )
            inv_rms = lax.rsqrt(ssq / d + 1e-6)
            for qq in range(N_Q):
                @pl.when(q == qq)
                def _():
                    for quarter, dst in (
                        (mine, qq),
                        (from_x, qq ^ 2),
                        (from_y, qq ^ 1),
                        (diag, qq ^ 3),
                    ):
                        sl = slice(dst * dq, (dst + 1) * dq)
                        out_ref[pl.ds(r * mr, mr), sl] = (
                            quarter * inv_rms * g_ref[:, sl]
                        )

        ring_round(0)
        ag0_x, ag0_y = ag_swap(0)
        ag0_x.start()
        ag0_y.start()
        ring_round(1)
        ag0_x.wait()
        ag0_y.wait()
        ag0_dy, ag0_dx = ag_diag(0)
        ag1_x, ag1_y = ag_swap(1)
        ag0_dy.start()
        ag0_dx.start()
        ag1_x.start()
        ag1_y.start()
        ag0_dy.wait()
        ag0_dx.wait()
        ag1_x.wait()
        ag1_y.wait()
        ag1_dy, ag1_dx = ag_diag(1)
        ag1_dy.start()
        ag1_dx.start()
        assemble(0)
        ag1_dy.wait()
        ag1_dx.wait()
        assemble(1)

    return pl.pallas_call(
        body,
        out_shape=jax.ShapeDtypeStruct((m_per, d), jnp.float32),
        in_specs=[
            pl.BlockSpec(memory_space=pltpu.VMEM),
            pl.BlockSpec(memory_space=pltpu.VMEM),
        ],
        out_specs=pl.BlockSpec(memory_space=pltpu.VMEM),
        scratch_shapes=[
            pltpu.VMEM((m_total, dq), jnp.float32),
            pltpu.VMEM((N_R, 2, mh, dq), jnp.float32),
            pltpu.VMEM((N_R, 2, mh, dq), jnp.float32),
            pltpu.VMEM((N_R, mr, dq), jnp.float32),
            pltpu.VMEM((N_R, 3, mr, dq), jnp.float32),
            pltpu.SemaphoreType.DMA((N_R, 2)),
            pltpu.SemaphoreType.DMA((N_R, 2)),
            pltpu.SemaphoreType.DMA((N_R, 2)),
            pltpu.SemaphoreType.DMA((N_R, 2)),
            pltpu.SemaphoreType.DMA((N_R, 4)),
            pltpu.SemaphoreType.DMA((N_R, 4)),
        ],
        compiler_params=pltpu.CompilerParams(collective_id=0),
    )(x, g)


# device time: 70219 ns/iter; 2.1965x vs baseline; 1.0543x over previous
import jax
import jax.numpy as jnp
from jax import lax
from jax.experimental import pallas as pl
from jax.experimental.pallas import tpu as pltpu

N_Z = 4
N_Q = 4
N_R = 2


def kernel(partial, gamma):
    _, m_total, d = partial.shape
    m_per = m_total // N_Z
    mr = m_per // N_R
    mh = mr // 2
    dq = d // N_Q
    x = jnp.reshape(partial, (m_total, d))
    g = jnp.reshape(gamma, (1, d))

    def body(
        x_ref,
        g_ref,
        out_ref,
        xq_ref,
        up_ref,
        dn_ref,
        agg_ref,
        gx_ref,
        up_send_sems,
        up_recv_sems,
        dn_send_sems,
        dn_recv_sems,
        ag_send_sems,
        ag_recv_sems,
    ):
        my_x = lax.axis_index("x")
        my_y = lax.axis_index("y")
        my_z = lax.axis_index("z")
        up = lax.rem(my_z + 1, N_Z)
        down = lax.rem(my_z + N_Z - 1, N_Z)
        q = my_x * 2 + my_y
        x_nbr = (1 - my_x, my_y, my_z)
        y_nbr = (my_x, 1 - my_y, my_z)

        partners = ((my_x, my_y, up), (my_x, my_y, down), x_nbr, y_nbr)
        barrier_sem = pltpu.get_barrier_semaphore()
        for dev in partners:
            pl.semaphore_signal(
                barrier_sem,
                inc=1,
                device_id=dev,
                device_id_type=pl.DeviceIdType.MESH,
            )

        for qq in range(N_Q):
            @pl.when(q == qq)
            def _():
                xq_ref[...] = x_ref[:, qq * dq:(qq + 1) * dq]
        c_up0 = lax.rem(my_z + N_Z - 1, N_Z)
        c_dn0 = lax.rem(my_z + 1, N_Z)
        for r in range(N_R):
            up_ref[r, 0] = xq_ref[pl.ds(c_up0 * m_per + r * mr, mh), :]
            dn_ref[r, 0] = xq_ref[pl.ds(c_dn0 * m_per + r * mr + mh, mh), :]

        pl.semaphore_wait(barrier_sem, len(partners))

        def ring_rdmas(r, s):
            send_slot = s % 2
            recv_slot = (s + 1) % 2
            rdma_up = pltpu.make_async_remote_copy(
                src_ref=up_ref.at[r, send_slot],
                dst_ref=up_ref.at[r, recv_slot],
                send_sem=up_send_sems.at[r, send_slot],
                recv_sem=up_recv_sems.at[r, recv_slot],
                device_id=(my_x, my_y, up),
                device_id_type=pl.DeviceIdType.MESH,
            )
            rdma_dn = pltpu.make_async_remote_copy(
                src_ref=dn_ref.at[r, send_slot],
                dst_ref=dn_ref.at[r, recv_slot],
                send_sem=dn_send_sems.at[r, send_slot],
                recv_sem=dn_recv_sems.at[r, recv_slot],
                device_id=(my_x, my_y, down),
                device_id_type=pl.DeviceIdType.MESH,
            )
            return rdma_up, rdma_dn

        def ring_compute(r, s):
            recv_slot = (s + 1) % 2
            c_up = lax.rem(my_z + 2 * N_Z - 2 - s, N_Z)
            c_dn = lax.rem(my_z + 2 + s, N_Z)
            acc_up = up_ref[r, recv_slot] + xq_ref[
                pl.ds(c_up * m_per + r * mr, mh), :
            ]
            acc_dn = dn_ref[r, recv_slot] + xq_ref[
                pl.ds(c_dn * m_per + r * mr + mh, mh), :
            ]
            if s < N_Z - 2:
                up_ref[r, recv_slot] = acc_up
                dn_ref[r, recv_slot] = acc_dn
            else:
                agg_ref[r, pl.ds(0, mh), :] = acc_up
                agg_ref[r, pl.ds(mh, mh), :] = acc_dn

        def ag_swap(r):
            ag_x = pltpu.make_async_remote_copy(
                src_ref=agg_ref.at[r],
                dst_ref=gx_ref.at[r, 0],
                send_sem=ag_send_sems.at[r, 0],
                recv_sem=ag_recv_sems.at[r, 0],
                device_id=x_nbr,
                device_id_type=pl.DeviceIdType.MESH,
            )
            ag_y = pltpu.make_async_remote_copy(
                src_ref=agg_ref.at[r],
                dst_ref=gx_ref.at[r, 1],
                send_sem=ag_send_sems.at[r, 1],
                recv_sem=ag_recv_sems.at[r, 1],
                device_id=y_nbr,
                device_id_type=pl.DeviceIdType.MESH,
            )
            return ag_x, ag_y

        def ag_diag(r):
            d_y = pltpu.make_async_remote_copy(
                src_ref=gx_ref.at[r, 0, pl.ds(0, mh)],
                dst_ref=gx_ref.at[r, 2, pl.ds(0, mh)],
                send_sem=ag_send_sems.at[r, 2],
                recv_sem=ag_recv_sems.at[r, 2],
                device_id=y_nbr,
                device_id_type=pl.DeviceIdType.MESH,
            )
            d_x = pltpu.make_async_remote_copy(
                src_ref=gx_ref.at[r, 1, pl.ds(mh, mh)],
                dst_ref=gx_ref.at[r, 2, pl.ds(mh, mh)],
                send_sem=ag_send_sems.at[r, 3],
                recv_sem=ag_recv_sems.at[r, 3],
                device_id=x_nbr,
                device_id_type=pl.DeviceIdType.MESH,
            )
            return d_y, d_x

        def row_ssq(v):
            return jnp.sum(v * v, axis=-1, keepdims=True)

        def assemble(r, ssq):
            inv_rms = lax.rsqrt(ssq / d + 1e-6)
            for qq in range(N_Q):
                @pl.when(q == qq)
                def _():
                    for quarter, dst in (
                        (agg_ref[r], qq),
                        (gx_ref[r, 0], qq ^ 2),
                        (gx_ref[r, 1], qq ^ 1),
                        (gx_ref[r, 2], qq ^ 3),
                    ):
                        sl = slice(dst * dq, (dst + 1) * dq)
                        out_ref[pl.ds(r * mr, mr), sl] = (
                            quarter * inv_rms * g_ref[:, sl]
                        )

        a0u, a0d = ring_rdmas(0, 0)
        a0u.start()
        a0d.start()
        a0u.wait()
        a0d.wait()
        ring_compute(0, 0)

        a1u, a1d = ring_rdmas(0, 1)
        b0u, b0d = ring_rdmas(1, 0)
        a1u.start()
        a1d.start()
        b0u.start()
        b0d.start()
        a1u.wait()
        a1d.wait()
        b0u.wait()
        b0d.wait()
        ring_compute(0, 1)
        ring_compute(1, 0)

        a2u, a2d = ring_rdmas(0, 2)
        b1u, b1d = ring_rdmas(1, 1)
        a2u.start()
        a2d.start()
        b1u.start()
        b1d.start()
        a2u.wait()
        a2d.wait()
        ring_compute(0, 2)
        ag0_x, ag0_y = ag_swap(0)
        ag0_x.start()
        ag0_y.start()
        b1u.wait()
        b1d.wait()
        ring_compute(1, 1)

        b2u, b2d = ring_rdmas(1, 2)
        b2u.start()
        b2d.start()
        b2u.wait()
        b2d.wait()
        ring_compute(1, 2)

        ag0_x.wait()
        ag0_y.wait()
        ag0_dy, ag0_dx = ag_diag(0)
        ag1_x, ag1_y = ag_swap(1)
        ag0_dy.start()
        ag0_dx.start()
        ag1_x.start()
        ag1_y.start()
        ssq0_part = (
            row_ssq(agg_ref[0]) + row_ssq(gx_ref[0, 0]) + row_ssq(gx_ref[0, 1])
        )
        ssq1_mine = row_ssq(agg_ref[1])
        ag0_dy.wait()
        ag0_dx.wait()
        ag1_x.wait()
        ag1_y.wait()
        ag1_dy, ag1_dx = ag_diag(1)
        ag1_dy.start()
        ag1_dx.start()
        assemble(0, ssq0_part + row_ssq(gx_ref[0, 2]))
        ag1_dy.wait()
        ag1_dx.wait()
        assemble(
            1,
            ssq1_mine
            + row_ssq(gx_ref[1, 0])
            + row_ssq(gx_ref[1, 1])
            + row_ssq(gx_ref[1, 2]),
        )

    return pl.pallas_call(
        body,
        out_shape=jax.ShapeDtypeStruct((m_per, d), jnp.float32),
        in_specs=[
            pl.BlockSpec(memory_space=pltpu.VMEM),
            pl.BlockSpec(memory_space=pltpu.VMEM),
        ],
        out_specs=pl.BlockSpec(memory_space=pltpu.VMEM),
        scratch_shapes=[
            pltpu.VMEM((m_total, dq), jnp.float32),
            pltpu.VMEM((N_R, 2, mh, dq), jnp.float32),
            pltpu.VMEM((N_R, 2, mh, dq), jnp.float32),
            pltpu.VMEM((N_R, mr, dq), jnp.float32),
            pltpu.VMEM((N_R, 3, mr, dq), jnp.float32),
            pltpu.SemaphoreType.DMA((N_R, 2)),
            pltpu.SemaphoreType.DMA((N_R, 2)),
            pltpu.SemaphoreType.DMA((N_R, 2)),
            pltpu.SemaphoreType.DMA((N_R, 2)),
            pltpu.SemaphoreType.DMA((N_R, 4)),
            pltpu.SemaphoreType.DMA((N_R, 4)),
        ],
        compiler_params=pltpu.CompilerParams(collective_id=0),
    )(x, g)


# device time: 64586 ns/iter; 2.3881x vs baseline; 1.0872x over previous
import jax
import jax.numpy as jnp
from jax import lax
from jax.experimental import pallas as pl
from jax.experimental.pallas import tpu as pltpu

N_Z = 4
N_Q = 4
N_R = 2


def kernel(partial, gamma):
    _, m_total, d = partial.shape
    m_per = m_total // N_Z
    mr = m_per // N_R
    mh = mr // 2
    dq = d // N_Q
    x = jnp.reshape(partial, (m_total, d))
    g = jnp.reshape(gamma, (1, d))

    def body(
        x_ref,
        g_ref,
        out_ref,
        xq_ref,
        up_ref,
        dn_ref,
        agg_ref,
        gx_ref,
        up_send_sems,
        up_recv_sems,
        dn_send_sems,
        dn_recv_sems,
        ag_send_sems,
        ag_recv_sems,
    ):
        my_x = lax.axis_index("x")
        my_y = lax.axis_index("y")
        my_z = lax.axis_index("z")
        up = lax.rem(my_z + 1, N_Z)
        down = lax.rem(my_z + N_Z - 1, N_Z)
        q = my_x * 2 + my_y
        x_nbr = (1 - my_x, my_y, my_z)
        y_nbr = (my_x, 1 - my_y, my_z)

        partners = ((my_x, my_y, up), (my_x, my_y, down), x_nbr, y_nbr)
        barrier_sem = pltpu.get_barrier_semaphore()
        for dev in partners:
            pl.semaphore_signal(
                barrier_sem,
                inc=1,
                device_id=dev,
                device_id_type=pl.DeviceIdType.MESH,
            )

        for qq in range(N_Q):
            @pl.when(q == qq)
            def _():
                xq_ref[...] = x_ref[:, qq * dq:(qq + 1) * dq]
        c_up0 = lax.rem(my_z + N_Z - 1, N_Z)
        c_dn0 = lax.rem(my_z + 1, N_Z)
        for r in range(N_R):
            up_ref[r, 0] = xq_ref[pl.ds(c_up0 * m_per + r * mr, mh), :]
            dn_ref[r, 0] = xq_ref[pl.ds(c_dn0 * m_per + r * mr + mh, mh), :]

        pl.semaphore_wait(barrier_sem, len(partners))

        def ring_rdmas(r, s):
            send_slot = s % 2
            recv_slot = (s + 1) % 2
            rdma_up = pltpu.make_async_remote_copy(
                src_ref=up_ref.at[r, send_slot],
                dst_ref=up_ref.at[r, recv_slot],
                send_sem=up_send_sems.at[r, send_slot],
                recv_sem=up_recv_sems.at[r, recv_slot],
                device_id=(my_x, my_y, up),
                device_id_type=pl.DeviceIdType.MESH,
            )
            rdma_dn = pltpu.make_async_remote_copy(
                src_ref=dn_ref.at[r, send_slot],
                dst_ref=dn_ref.at[r, recv_slot],
                send_sem=dn_send_sems.at[r, send_slot],
                recv_sem=dn_recv_sems.at[r, recv_slot],
                device_id=(my_x, my_y, down),
                device_id_type=pl.DeviceIdType.MESH,
            )
            return rdma_up, rdma_dn

        def ring_compute(r, s):
            recv_slot = (s + 1) % 2
            c_up = lax.rem(my_z + 2 * N_Z - 2 - s, N_Z)
            c_dn = lax.rem(my_z + 2 + s, N_Z)
            acc_up = up_ref[r, recv_slot] + xq_ref[
                pl.ds(c_up * m_per + r * mr, mh), :
            ]
            acc_dn = dn_ref[r, recv_slot] + xq_ref[
                pl.ds(c_dn * m_per + r * mr + mh, mh), :
            ]
            if s < N_Z - 2:
                up_ref[r, recv_slot] = acc_up
                dn_ref[r, recv_slot] = acc_dn
            else:
                agg_ref[r, pl.ds(0, mh), :] = acc_up
                agg_ref[r, pl.ds(mh, mh), :] = acc_dn

        def ag_swap(r):
            ag_x = pltpu.make_async_remote_copy(
                src_ref=agg_ref.at[r],
                dst_ref=gx_ref.at[r, 0],
                send_sem=ag_send_sems.at[r, 0],
                recv_sem=ag_recv_sems.at[r, 0],
                device_id=x_nbr,
                device_id_type=pl.DeviceIdType.MESH,
            )
            ag_y = pltpu.make_async_remote_copy(
                src_ref=agg_ref.at[r],
                dst_ref=gx_ref.at[r, 1],
                send_sem=ag_send_sems.at[r, 1],
                recv_sem=ag_recv_sems.at[r, 1],
                device_id=y_nbr,
                device_id_type=pl.DeviceIdType.MESH,
            )
            return ag_x, ag_y

        def ag_diag(r):
            d_y = pltpu.make_async_remote_copy(
                src_ref=gx_ref.at[r, 0, pl.ds(0, mh)],
                dst_ref=gx_ref.at[r, 2, pl.ds(0, mh)],
                send_sem=ag_send_sems.at[r, 2],
                recv_sem=ag_recv_sems.at[r, 2],
                device_id=y_nbr,
                device_id_type=pl.DeviceIdType.MESH,
            )
            d_x = pltpu.make_async_remote_copy(
                src_ref=gx_ref.at[r, 1, pl.ds(mh, mh)],
                dst_ref=gx_ref.at[r, 2, pl.ds(mh, mh)],
                send_sem=ag_send_sems.at[r, 3],
                recv_sem=ag_recv_sems.at[r, 3],
                device_id=x_nbr,
                device_id_type=pl.DeviceIdType.MESH,
            )
            return d_y, d_x

        def row_ssq(v):
            return jnp.sum(v * v, axis=-1, keepdims=True)

        def assemble(r, ssq):
            inv_rms = lax.rsqrt(ssq / d + 1e-6)
            for qq in range(N_Q):
                @pl.when(q == qq)
                def _():
                    for quarter, dst in (
                        (agg_ref[r], qq),
                        (gx_ref[r, 0], qq ^ 2),
                        (gx_ref[r, 1], qq ^ 1),
                        (gx_ref[r, 2], qq ^ 3),
                    ):
                        sl = slice(dst * dq, (dst + 1) * dq)
                        out_ref[pl.ds(r * mr, mr), sl] = (
                            quarter * inv_rms * g_ref[:, sl]
                        )

        a0u, a0d = ring_rdmas(0, 0)
        b0u, b0d = ring_rdmas(1, 0)
        a0u.start()
        a0d.start()
        b0u.start()
        b0d.start()

        a0u.wait()
        a0d.wait()
        ring_compute(0, 0)
        a1u, a1d = ring_rdmas(0, 1)
        a1u.start()
        a1d.start()

        b0u.wait()
        b0d.wait()
        ring_compute(1, 0)
        b1u, b1d = ring_rdmas(1, 1)
        b1u.start()
        b1d.start()

        a1u.wait()
        a1d.wait()
        ring_compute(0, 1)
        a2u, a2d = ring_rdmas(0, 2)
        a2u.start()
        a2d.start()

        b1u.wait()
        b1d.wait()
        ring_compute(1, 1)
        b2u, b2d = ring_rdmas(1, 2)
        b2u.start()
        b2d.start()

        a2u.wait()
        a2d.wait()
        ring_compute(0, 2)
        ag0_x, ag0_y = ag_swap(0)
        ag0_x.start()
        ag0_y.start()

        b2u.wait()
        b2d.wait()
        ring_compute(1, 2)

        ag0_x.wait()
        ag0_y.wait()
        ag0_dy, ag0_dx = ag_diag(0)
        ag1_x, ag1_y = ag_swap(1)
        ag0_dy.start()
        ag0_dx.start()
        ag1_x.start()
        ag1_y.start()
        ssq0_part = (
            row_ssq(agg_ref[0]) + row_ssq(gx_ref[0, 0]) + row_ssq(gx_ref[0, 1])
        )
        ssq1_mine = row_ssq(agg_ref[1])
        ag0_dy.wait()
        ag0_dx.wait()
        ag1_x.wait()
        ag1_y.wait()
        ag1_dy, ag1_dx = ag_diag(1)
        ag1_dy.start()
        ag1_dx.start()
        assemble(0, ssq0_part + row_ssq(gx_ref[0, 2]))
        ag1_dy.wait()
        ag1_dx.wait()
        assemble(
            1,
            ssq1_mine
            + row_ssq(gx_ref[1, 0])
            + row_ssq(gx_ref[1, 1])
            + row_ssq(gx_ref[1, 2]),
        )

    return pl.pallas_call(
        body,
        out_shape=jax.ShapeDtypeStruct((m_per, d), jnp.float32),
        in_specs=[
            pl.BlockSpec(memory_space=pltpu.VMEM),
            pl.BlockSpec(memory_space=pltpu.VMEM),
        ],
        out_specs=pl.BlockSpec(memory_space=pltpu.VMEM),
        scratch_shapes=[
            pltpu.VMEM((m_total, dq), jnp.float32),
            pltpu.VMEM((N_R, 2, mh, dq), jnp.float32),
            pltpu.VMEM((N_R, 2, mh, dq), jnp.float32),
            pltpu.VMEM((N_R, mr, dq), jnp.float32),
            pltpu.VMEM((N_R, 3, mr, dq), jnp.float32),
            pltpu.SemaphoreType.DMA((N_R, 2)),
            pltpu.SemaphoreType.DMA((N_R, 2)),
            pltpu.SemaphoreType.DMA((N_R, 2)),
            pltpu.SemaphoreType.DMA((N_R, 2)),
            pltpu.SemaphoreType.DMA((N_R, 4)),
            pltpu.SemaphoreType.DMA((N_R, 4)),
        ],
        compiler_params=pltpu.CompilerParams(collective_id=0),
    )(x, g)
